# baseline (speedup 1.0000x reference)
import functools

import jax
import jax.numpy as jnp
from jax import lax
from jax.experimental import pallas as pl
from jax.experimental.pallas import tpu as pltpu
from jax.experimental.pallas import tpu_sc as plsc

NC = 2
NS = 16
NW = NC * NS
XB = 8192
QB = XB // 4


def _pack_bf16(t):
    r = lax.bitcast_convert_type(t, jnp.uint32)
    hi = r[:, :32]
    lo = r[:, 32:] >> 16
    return lax.bitcast_convert_type(hi | lo, jnp.float32)


def _xpose_body(eye_ref, inT_ref, out_ref):
    x = inT_ref[...]
    xb = x.astype(jnp.bfloat16)
    eye = eye_ref[...]
    cdim = (((0,), (0,)), ((), ()))
    parts = []
    for q in range(4):
        t = lax.dot_general(xb[:, q * QB:(q + 1) * QB], eye, cdim,
                            preferred_element_type=jnp.float32)
        parts.append(_pack_bf16(t))
    y = jnp.concatenate(parts, axis=1)
    out_ref[...] = y.reshape(QB * 128)


@functools.cache
def _make_tc_xpose(V, D):
    grid = (V + XB - 1) // XB
    return pl.pallas_call(
        _xpose_body,
        grid=(grid,),
        in_specs=[pl.BlockSpec((D, D), lambda i: (0, 0)),
                  pl.BlockSpec((D, XB), lambda i: (0, i))],
        out_specs=pl.BlockSpec((QB * 128,), lambda i: (i,)),
        out_shape=jax.ShapeDtypeStruct((grid * QB * 128,), jnp.float32),
    )


@functools.cache
def _make_sc_gather(B, M):
    BPW = B // NW
    mesh = plsc.VectorSubcoreMesh(core_axis_name="c", subcore_axis_name="s")

    @functools.partial(
        pl.kernel,
        mesh=mesh,
        compiler_params=pltpu.CompilerParams(use_tc_tiling_on_sc=False),
        out_type=(
            jax.ShapeDtypeStruct((B * 32,), jnp.float32),
            jax.ShapeDtypeStruct((B * 32,), jnp.float32),
        ),
        scratch_types=[
            pltpu.VMEM((BPW,), jnp.int32),
            pltpu.VMEM((BPW,), jnp.int32),
            pltpu.VMEM((BPW * 32,), jnp.float32),
            pltpu.VMEM((BPW * 32,), jnp.float32),
            pltpu.SemaphoreType.DMA,
            pltpu.SemaphoreType.DMA,
        ],
    )
    def sc_gather(woffu_hbm, woffi_hbm, utab_hbm, itab_hbm, ue_hbm, ie_hbm,
                  uoff_v, ioff_v, ubuf_v, ibuf_v, sem_u, sem_i):
        wid = lax.axis_index("s") * NC + lax.axis_index("c")
        base = wid * BPW
        pltpu.sync_copy(woffu_hbm.at[pl.ds(base, BPW)], uoff_v)
        pltpu.sync_copy(woffi_hbm.at[pl.ds(base, BPW)], ioff_v)

        def body_u(c, carry):
            s0 = c * 16
            uvec = uoff_v[pl.ds(s0, 16)]
            for j in range(16):
                off = pl.multiple_of(uvec[j], 32)
                pltpu.async_copy(utab_hbm.at[pl.ds(off, 32)],
                                 ubuf_v.at[pl.ds((s0 + j) * 32, 32)], sem_u)
            return carry

        def body_i(c, carry):
            s0 = c * 16
            ivec = ioff_v[pl.ds(s0, 16)]
            for j in range(16):
                off = pl.multiple_of(ivec[j], 32)
                pltpu.async_copy(itab_hbm.at[pl.ds(off, 32)],
                                 ibuf_v.at[pl.ds((s0 + j) * 32, 32)], sem_i)
            return carry

        lax.fori_loop(0, BPW // 16, body_u, 0)
        lax.fori_loop(0, BPW // 16, body_i, 0)
        pltpu.make_async_copy(utab_hbm.at[pl.ds(0, BPW * 32)], ubuf_v,
                              sem_u).wait()
        pltpu.make_async_copy(itab_hbm.at[pl.ds(0, BPW * 32)], ibuf_v,
                              sem_i).wait()
        pltpu.sync_copy(ubuf_v, ue_hbm.at[pl.ds(base * 32, BPW * 32)])
        pltpu.sync_copy(ibuf_v, ie_hbm.at[pl.ds(base * 32, BPW * 32)])

    return sc_gather


def _unpack_bf16(w32):
    r = lax.bitcast_convert_type(w32, jnp.uint32)
    hi = lax.bitcast_convert_type(r & jnp.uint32(0xFFFF0000), jnp.float32)
    lo = lax.bitcast_convert_type(r << 16, jnp.float32)
    return jnp.concatenate([hi, lo], axis=1)


def _dense_body(uew_ref, iew_ref, uf_ref, wuf_ref, buf_ref, wt_ref, bt_ref,
                out_ref, *, DU):
    ue = _unpack_bf16(uew_ref[...])
    ie = _unpack_bf16(iew_ref[...])
    uft = jnp.dot(uf_ref[...], wuf_ref[...],
                  preferred_element_type=jnp.float32) + buf_ref[...]
    wt = wt_ref[...]
    acc = jnp.dot(ue, wt[0:DU], preferred_element_type=jnp.float32)
    acc = acc + jnp.dot(uft, wt[DU:2 * DU], preferred_element_type=jnp.float32)
    acc = acc + jnp.dot(ie, wt[2 * DU:], preferred_element_type=jnp.float32)
    out_ref[...] = acc + bt_ref[...]


@functools.cache
def _make_tc_dense(B, DU, DI, IU, T, BLK=2048):
    grid = B // BLK
    return pl.pallas_call(
        functools.partial(_dense_body, DU=DU),
        grid=(grid,),
        in_specs=[
            pl.BlockSpec((BLK, 32), lambda i: (i, 0)),
            pl.BlockSpec((BLK, 32), lambda i: (i, 0)),
            pl.BlockSpec((BLK, IU), lambda i: (i, 0)),
            pl.BlockSpec((IU, DU), lambda i: (0, 0)),
            pl.BlockSpec((1, DU), lambda i: (0, 0)),
            pl.BlockSpec((2 * DU + DI, T), lambda i: (0, 0)),
            pl.BlockSpec((1, T), lambda i: (0, 0)),
        ],
        out_specs=pl.BlockSpec((BLK, T), lambda i: (i, 0)),
        out_shape=jax.ShapeDtypeStruct((B, T), jnp.float32),
    )


def _word_offsets(idx):
    step = idx // XB
    q = (idx // QB) & 3
    j = idx & (QB - 1)
    return (step * QB + j) * 128 + q * 32


def kernel(user_id, user_features, item_id, user_table, item_table,
           W_uf, b_uf, W_task, b_task):
    B = user_id.shape[0]
    VU, DU = user_table.shape
    VI, DI = item_table.shape
    IU = user_features.shape[1]
    T = W_task.shape[1]
    uid = user_id.astype(jnp.int32)
    iid = item_id.astype(jnp.int32)
    eye = jnp.eye(DU, dtype=jnp.bfloat16)
    upk = _make_tc_xpose(VU, DU)(eye, user_table.T)
    ipk = _make_tc_xpose(VI, DI)(eye, item_table.T)
    uew, iew = _make_sc_gather(B, upk.shape[0])(
        _word_offsets(uid), _word_offsets(iid), upk, ipk)
    return _make_tc_dense(B, DU, DI, IU, T)(
        uew.reshape(B, 32), iew.reshape(B, 32), user_features, W_uf,
        b_uf.reshape(1, DU), W_task, b_task.reshape(1, T))


# speedup vs baseline: 1.2276x; 1.2276x over previous
import functools

import jax
import jax.numpy as jnp
from jax import lax
from jax.experimental import pallas as pl
from jax.experimental.pallas import tpu as pltpu
from jax.experimental.pallas import tpu_sc as plsc

NC = 2
NS = 16
NW = NC * NS
XB = 8192
QB = XB // 4


def _xpose_body(inT_ref, out_ref):
    x = inT_ref[...]
    r = lax.bitcast_convert_type(x, jnp.uint32)
    r = (r + jnp.uint32(0x8000)) & jnp.uint32(0xFFFF0000)
    p = r[:32, :] | (r[32:, :] >> 16)
    pf = lax.bitcast_convert_type(p, jnp.float32)
    parts = [jnp.swapaxes(pf[:, q * QB:(q + 1) * QB], 0, 1)
             for q in range(4)]
    y = jnp.concatenate(parts, axis=1)
    out_ref[...] = y.reshape(QB * 128)


@functools.cache
def _make_tc_xpose(V, D):
    grid = (V + XB - 1) // XB
    return pl.pallas_call(
        _xpose_body,
        grid=(grid,),
        in_specs=[pl.BlockSpec((D, XB), lambda i: (0, i))],
        out_specs=pl.BlockSpec((QB * 128,), lambda i: (i,)),
        out_shape=jax.ShapeDtypeStruct((grid * QB * 128,), jnp.float32),
    )


@functools.cache
def _make_sc_gather(B, M):
    BPW = B // NW
    mesh = plsc.VectorSubcoreMesh(core_axis_name="c", subcore_axis_name="s")

    @functools.partial(
        pl.kernel,
        mesh=mesh,
        compiler_params=pltpu.CompilerParams(use_tc_tiling_on_sc=False),
        out_type=(
            jax.ShapeDtypeStruct((B * 32,), jnp.float32),
            jax.ShapeDtypeStruct((B * 32,), jnp.float32),
        ),
        scratch_types=[
            pltpu.VMEM((BPW,), jnp.int32),
            pltpu.VMEM((BPW,), jnp.int32),
            pltpu.VMEM((BPW * 32,), jnp.float32),
            pltpu.VMEM((BPW * 32,), jnp.float32),
            pltpu.SemaphoreType.DMA,
            pltpu.SemaphoreType.DMA,
        ],
    )
    def sc_gather(woffu_hbm, woffi_hbm, utab_hbm, itab_hbm, ue_hbm, ie_hbm,
                  uoff_v, ioff_v, ubuf_v, ibuf_v, sem_u, sem_i):
        wid = lax.axis_index("s") * NC + lax.axis_index("c")
        base = wid * BPW
        pltpu.sync_copy(woffu_hbm.at[pl.ds(base, BPW)], uoff_v)
        pltpu.sync_copy(woffi_hbm.at[pl.ds(base, BPW)], ioff_v)

        def body_u(c, carry):
            s0 = c * 16
            uvec = uoff_v[pl.ds(s0, 16)]
            for j in range(16):
                off = pl.multiple_of(uvec[j], 32)
                pltpu.async_copy(utab_hbm.at[pl.ds(off, 32)],
                                 ubuf_v.at[pl.ds((s0 + j) * 32, 32)], sem_u)
            return carry

        def body_i(c, carry):
            s0 = c * 16
            ivec = ioff_v[pl.ds(s0, 16)]
            for j in range(16):
                off = pl.multiple_of(ivec[j], 32)
                pltpu.async_copy(itab_hbm.at[pl.ds(off, 32)],
                                 ibuf_v.at[pl.ds((s0 + j) * 32, 32)], sem_i)
            return carry

        lax.fori_loop(0, BPW // 16, body_u, 0)
        lax.fori_loop(0, BPW // 16, body_i, 0)
        pltpu.make_async_copy(utab_hbm.at[pl.ds(0, BPW * 32)], ubuf_v,
                              sem_u).wait()
        pltpu.make_async_copy(itab_hbm.at[pl.ds(0, BPW * 32)], ibuf_v,
                              sem_i).wait()
        pltpu.sync_copy(ubuf_v, ue_hbm.at[pl.ds(base * 32, BPW * 32)])
        pltpu.sync_copy(ibuf_v, ie_hbm.at[pl.ds(base * 32, BPW * 32)])

    return sc_gather


def _unpack_bf16(w32):
    r = lax.bitcast_convert_type(w32, jnp.uint32)
    hi = lax.bitcast_convert_type(r & jnp.uint32(0xFFFF0000), jnp.float32)
    lo = lax.bitcast_convert_type(r << 16, jnp.float32)
    return jnp.concatenate([hi, lo], axis=1)


def _dense_body(uew_ref, iew_ref, uf_ref, wuf_ref, buf_ref, wt_ref, bt_ref,
                out_ref, *, DU):
    ue = _unpack_bf16(uew_ref[...])
    ie = _unpack_bf16(iew_ref[...])
    uft = jnp.dot(uf_ref[...], wuf_ref[...],
                  preferred_element_type=jnp.float32) + buf_ref[...]
    wt = wt_ref[...]
    acc = jnp.dot(ue, wt[0:DU], preferred_element_type=jnp.float32)
    acc = acc + jnp.dot(uft, wt[DU:2 * DU], preferred_element_type=jnp.float32)
    acc = acc + jnp.dot(ie, wt[2 * DU:], preferred_element_type=jnp.float32)
    out_ref[...] = acc + bt_ref[...]


@functools.cache
def _make_tc_dense(B, DU, DI, IU, T, BLK=2048):
    grid = B // BLK
    return pl.pallas_call(
        functools.partial(_dense_body, DU=DU),
        grid=(grid,),
        in_specs=[
            pl.BlockSpec((BLK, 32), lambda i: (i, 0)),
            pl.BlockSpec((BLK, 32), lambda i: (i, 0)),
            pl.BlockSpec((BLK, IU), lambda i: (i, 0)),
            pl.BlockSpec((IU, DU), lambda i: (0, 0)),
            pl.BlockSpec((1, DU), lambda i: (0, 0)),
            pl.BlockSpec((2 * DU + DI, T), lambda i: (0, 0)),
            pl.BlockSpec((1, T), lambda i: (0, 0)),
        ],
        out_specs=pl.BlockSpec((BLK, T), lambda i: (i, 0)),
        out_shape=jax.ShapeDtypeStruct((B, T), jnp.float32),
    )


def _word_offsets(idx):
    step = idx // XB
    q = (idx // QB) & 3
    j = idx & (QB - 1)
    return (step * QB + j) * 128 + q * 32


def kernel(user_id, user_features, item_id, user_table, item_table,
           W_uf, b_uf, W_task, b_task):
    B = user_id.shape[0]
    VU, DU = user_table.shape
    VI, DI = item_table.shape
    IU = user_features.shape[1]
    T = W_task.shape[1]
    uid = user_id.astype(jnp.int32)
    iid = item_id.astype(jnp.int32)
    upk = _make_tc_xpose(VU, DU)(user_table.T)
    ipk = _make_tc_xpose(VI, DI)(item_table.T)
    uew, iew = _make_sc_gather(B, upk.shape[0])(
        _word_offsets(uid), _word_offsets(iid), upk, ipk)
    return _make_tc_dense(B, DU, DI, IU, T)(
        uew.reshape(B, 32), iew.reshape(B, 32), user_features, W_uf,
        b_uf.reshape(1, DU), W_task, b_task.reshape(1, T))
